# baseline (speedup 1.0000x reference)
import functools

import jax
import jax.numpy as jnp
from jax import lax
from jax.experimental import pallas as pl
from jax.experimental.pallas import tpu as pltpu
from jax.experimental.pallas import tpu_sc as plsc

_EMB = 1000000
_DIM = 64
_NCLS = 1000
_NIDS = 4096
_NIDS_SC = 3072
_NIDS_TC = _NIDS - _NIDS_SC
_NC = 2
_NS = 16
_NW = _NC * _NS
_PER_W = _NIDS_SC // _NW
_LANES = 16
_G = _DIM // _LANES
_BLK = 128
_NB = 8
_AHEAD = 6
_IPB = 64

_mesh = plsc.VectorSubcoreMesh(core_axis_name="c", subcore_axis_name="s")


@functools.partial(
    pl.kernel,
    mesh=_mesh,
    out_type=jax.ShapeDtypeStruct((_NW * _DIM,), jnp.float32),
    scratch_types=[
        pltpu.VMEM((_PER_W,), jnp.int32),
        pltpu.VMEM((_NB, _DIM, _BLK), jnp.float32),
        pltpu.VMEM((_DIM,), jnp.float32),
        [pltpu.SemaphoreType.DMA] * _NB,
    ],
    compiler_params=pltpu.CompilerParams(
        needs_layout_passes=False, disable_bounds_checks=True
    ),
)
def _gather_partial_sums(ids_hbm, tablet_hbm, out_hbm, idx_v, ring_v, acc_v, sems):
    wid = lax.axis_index("s") * _NC + lax.axis_index("c")
    base = pl.multiple_of(wid * _PER_W, _PER_W)
    obase = pl.multiple_of(wid * _DIM, _DIM)
    pltpu.sync_copy(ids_hbm.at[pl.ds(base, _PER_W)], idx_v)

    lane_iota = lax.iota(jnp.int32, _LANES)

    ids_s = []
    for k in range(_PER_W // _LANES):
        v = idx_v[pl.ds(k * _LANES, _LANES)]
        for j in range(_LANES):
            ids_s.append(jnp.sum(jnp.where(lane_iota == j, v, 0)))

    def issue(i):
        start = pl.multiple_of(ids_s[i] & ~(_BLK - 1), _BLK)
        return pltpu.async_copy(
            tablet_hbm.at[:, pl.ds(start, _BLK)], ring_v.at[i % _NB], sems[i % _NB]
        )

    handles = {}
    for i in range(_AHEAD):
        handles[i] = issue(i)

    acc = [jnp.zeros((_LANES,), jnp.float32) for _ in range(_G)]
    for i in range(_PER_W):
        if i + _AHEAD < _PER_W:
            handles[i + _AHEAD] = issue(i + _AHEAD)
        handles.pop(i).wait()
        bsplat = jnp.full((_LANES,), i % _NB, jnp.int32)
        lsplat = jnp.full((_LANES,), ids_s[i] & (_BLK - 1), jnp.int32)
        for g in range(_G):
            acc[g] = acc[g] + plsc.load_gather(
                ring_v, [bsplat, lane_iota + g * _LANES, lsplat]
            )

    for g in range(_G):
        acc_v[pl.ds(g * _LANES, _LANES)] = acc[g]
    pltpu.sync_copy(acc_v, out_hbm.at[pl.ds(obase, _DIM)])


def _tc_gather_body(ids_ref, *refs):
    out_ref = refs[_IPB]
    step = pl.program_id(0)

    @pl.when(step == 0)
    def _():
        out_ref[...] = jnp.zeros_like(out_ref)

    lane = lax.broadcasted_iota(jnp.int32, (1, _BLK), 1)
    acc = out_ref[...]
    for k in range(_IPB):
        the_id = ids_ref[_NIDS_SC + step * _IPB + k]
        mask = lane == (the_id & (_BLK - 1))
        acc = acc + jnp.where(mask, refs[k][...], 0.0)
    out_ref[...] = acc


_tc_gather = pl.pallas_call(
    _tc_gather_body,
    grid_spec=pltpu.PrefetchScalarGridSpec(
        num_scalar_prefetch=1,
        grid=(_NIDS_TC // _IPB,),
        in_specs=[
            pl.BlockSpec(
                (_DIM, _BLK),
                functools.partial(
                    lambda k, i, ids_ref: (0, ids_ref[_NIDS_SC + i * _IPB + k] // _BLK),
                    k,
                ),
            )
            for k in range(_IPB)
        ],
        out_specs=pl.BlockSpec((_DIM, _BLK), lambda i, ids_ref: (0, 0)),
    ),
    out_shape=jax.ShapeDtypeStruct((_DIM, _BLK), jnp.float32),
    compiler_params=pltpu.CompilerParams(
        dimension_semantics=("arbitrary",)
    ),
)


def _classifier_body(p_ref, t_ref, w_ref, b_ref, o_ref):
    p = p_ref[...]
    s = p[:, 0:_DIM]
    for w in range(1, _NW):
        s = s + p[:, w * _DIM:(w + 1) * _DIM]
    scale = 1.0 / _NIDS
    logits_sc = jnp.dot(
        s * scale, w_ref[...], preferred_element_type=jnp.float32
    )
    tc_col = jnp.sum(t_ref[...], axis=1, keepdims=True) * scale
    logits_tc = lax.dot_general(
        tc_col, w_ref[...], (((0,), (0,)), ((), ())),
        preferred_element_type=jnp.float32,
    )
    o_ref[...] = logits_sc + logits_tc + b_ref[...]


_classifier = pl.pallas_call(
    _classifier_body,
    out_shape=jax.ShapeDtypeStruct((1, _NCLS), jnp.float32),
)


def kernel(ids, embedding, W, b):
    ids32 = ids.astype(jnp.int32)
    tablet = embedding.T
    partials = _gather_partial_sums(ids32, tablet)
    tc_part = _tc_gather(ids32, *([tablet] * _IPB))
    logits = _classifier(
        partials.reshape(1, _NW * _DIM), tc_part, W, b.reshape(1, _NCLS)
    )
    return logits[0]


# speedup vs baseline: 1.0163x; 1.0163x over previous
import functools

import jax
import jax.numpy as jnp
from jax import lax
from jax.experimental import pallas as pl
from jax.experimental.pallas import tpu as pltpu
from jax.experimental.pallas import tpu_sc as plsc

_EMB = 1000000
_DIM = 64
_NCLS = 1000
_NIDS = 4096
_NIDS_SC = 3072
_NIDS_TC = _NIDS - _NIDS_SC
_NC = 2
_NS = 16
_NW = _NC * _NS
_PER_W = _NIDS_SC // _NW
_LANES = 16
_G = _DIM // _LANES
_BLK = 128
_NB = 8
_AHEAD = 6
_IPB = 32

_mesh = plsc.VectorSubcoreMesh(core_axis_name="c", subcore_axis_name="s")


@functools.partial(
    pl.kernel,
    mesh=_mesh,
    out_type=jax.ShapeDtypeStruct((_NW * _DIM,), jnp.float32),
    scratch_types=[
        pltpu.VMEM((_PER_W,), jnp.int32),
        pltpu.VMEM((_NB, _DIM, _BLK), jnp.float32),
        pltpu.VMEM((_DIM,), jnp.float32),
        [pltpu.SemaphoreType.DMA] * _NB,
    ],
    compiler_params=pltpu.CompilerParams(
        needs_layout_passes=False, disable_bounds_checks=True
    ),
)
def _gather_partial_sums(ids_hbm, tablet_hbm, out_hbm, idx_v, ring_v, acc_v, sems):
    wid = lax.axis_index("s") * _NC + lax.axis_index("c")
    base = pl.multiple_of(wid * _PER_W, _PER_W)
    obase = pl.multiple_of(wid * _DIM, _DIM)
    pltpu.sync_copy(ids_hbm.at[pl.ds(base, _PER_W)], idx_v)

    lane_iota = lax.iota(jnp.int32, _LANES)

    ids_s = []
    for k in range(_PER_W // _LANES):
        v = idx_v[pl.ds(k * _LANES, _LANES)]
        for j in range(_LANES):
            ids_s.append(jnp.sum(jnp.where(lane_iota == j, v, 0)))

    def issue(i):
        start = pl.multiple_of(ids_s[i] & ~(_BLK - 1), _BLK)
        return pltpu.async_copy(
            tablet_hbm.at[:, pl.ds(start, _BLK)], ring_v.at[i % _NB], sems[i % _NB]
        )

    handles = {}
    for i in range(_AHEAD):
        handles[i] = issue(i)

    acc = [jnp.zeros((_LANES,), jnp.float32) for _ in range(_G)]
    for i in range(_PER_W):
        if i + _AHEAD < _PER_W:
            handles[i + _AHEAD] = issue(i + _AHEAD)
        handles.pop(i).wait()
        bsplat = jnp.full((_LANES,), i % _NB, jnp.int32)
        lsplat = jnp.full((_LANES,), ids_s[i] & (_BLK - 1), jnp.int32)
        for g in range(_G):
            acc[g] = acc[g] + plsc.load_gather(
                ring_v, [bsplat, lane_iota + g * _LANES, lsplat]
            )

    for g in range(_G):
        acc_v[pl.ds(g * _LANES, _LANES)] = acc[g]
    pltpu.sync_copy(acc_v, out_hbm.at[pl.ds(obase, _DIM)])


def _tc_gather_body(ids_ref, *refs):
    out_ref = refs[_IPB]
    step = pl.program_id(0)

    @pl.when(step == 0)
    def _():
        out_ref[...] = jnp.zeros_like(out_ref)

    lane = lax.broadcasted_iota(jnp.int32, (1, _BLK), 1)
    acc = out_ref[...]
    for k in range(_IPB):
        the_id = ids_ref[_NIDS_SC + step * _IPB + k]
        mask = lane == (the_id & (_BLK - 1))
        acc = acc + jnp.where(mask, refs[k][...], 0.0)
    out_ref[...] = acc


_tc_gather = pl.pallas_call(
    _tc_gather_body,
    grid_spec=pltpu.PrefetchScalarGridSpec(
        num_scalar_prefetch=1,
        grid=(_NIDS_TC // _IPB,),
        in_specs=[
            pl.BlockSpec(
                (_DIM, _BLK),
                functools.partial(
                    lambda k, i, ids_ref: (0, ids_ref[_NIDS_SC + i * _IPB + k] // _BLK),
                    k,
                ),
            )
            for k in range(_IPB)
        ],
        out_specs=pl.BlockSpec((_DIM, _BLK), lambda i, ids_ref: (0, 0)),
    ),
    out_shape=jax.ShapeDtypeStruct((_DIM, _BLK), jnp.float32),
    compiler_params=pltpu.CompilerParams(
        dimension_semantics=("arbitrary",)
    ),
)


def _classifier_body(p_ref, t_ref, w_ref, b_ref, o_ref):
    p = p_ref[...]
    s = p[:, 0:_DIM]
    for w in range(1, _NW):
        s = s + p[:, w * _DIM:(w + 1) * _DIM]
    scale = 1.0 / _NIDS
    logits_sc = jnp.dot(
        s * scale, w_ref[...], preferred_element_type=jnp.float32
    )
    tc_col = jnp.sum(t_ref[...], axis=1, keepdims=True) * scale
    logits_tc = lax.dot_general(
        tc_col, w_ref[...], (((0,), (0,)), ((), ())),
        preferred_element_type=jnp.float32,
    )
    o_ref[...] = logits_sc + logits_tc + b_ref[...]


_classifier = pl.pallas_call(
    _classifier_body,
    out_shape=jax.ShapeDtypeStruct((1, _NCLS), jnp.float32),
)


def kernel(ids, embedding, W, b):
    ids32 = ids.astype(jnp.int32)
    tablet = embedding.T
    partials = _gather_partial_sums(ids32, tablet)
    tc_part = _tc_gather(ids32, *([tablet] * _IPB))
    logits = _classifier(
        partials.reshape(1, _NW * _DIM), tc_part, W, b.reshape(1, _NCLS)
    )
    return logits[0]
